# baseline (speedup 1.0000x reference)
import functools

import jax
import jax.numpy as jnp
from jax import lax
from jax.experimental import pallas as pl
from jax.experimental.pallas import tpu as pltpu
from jax.experimental.pallas import tpu_sc as plsc

N, E, G, D_IN, D_HID = 10000, 320000, 256, 128, 512

NC, NS = 2, 16
NW = NC * NS
CHUNK = 128
EPT = E // NW
NCHUNK = 80
PHCH = 40
IDXROWS = 48
NIDX = 88
EPT_PAD = NIDX * CHUNK
E_PAD = EPT_PAD * NW
N_ACC = 10112
NPS = N_ACC // NS


def _edge_agg_body(v_hbm, src_hbm, dst_hbm, zeros_hbm, out_hbm,
                   acc_shared, src_idx, dst_idx, rows0, rows1, sem0, sem1):
    c = lax.axis_index("c")
    s = lax.axis_index("s")
    wid = s * NC + c
    pltpu.sync_copy(zeros_hbm, acc_shared.at[pl.ds(s * NPS, NPS)])
    plsc.subcore_barrier()

    for p in range(2):
        pltpu.sync_copy(src_hbm.at[wid * 2 + p], src_idx)
        pltpu.sync_copy(dst_hbm.at[wid * 2 + p], dst_idx)
        pltpu.async_copy(v_hbm.at[src_idx.at[0]], rows0, sem0)

        def step(t, carry):
            j = 2 * t
            pltpu.make_async_copy(v_hbm.at[src_idx.at[j]], rows0, sem0).wait()
            pltpu.async_copy(v_hbm.at[src_idx.at[j + 1]], rows1, sem1)
            pltpu.sync_copy(rows0, acc_shared.at[dst_idx.at[j]], add=True)
            pltpu.make_async_copy(v_hbm.at[src_idx.at[j + 1]], rows1,
                                  sem1).wait()
            pltpu.async_copy(v_hbm.at[src_idx.at[j + 2]], rows0, sem0)
            pltpu.sync_copy(rows1, acc_shared.at[dst_idx.at[j + 1]], add=True)
            return carry

        lax.fori_loop(0, PHCH // 2, step, 0)
        pltpu.make_async_copy(v_hbm.at[src_idx.at[PHCH]], rows0, sem0).wait()

    plsc.subcore_barrier()
    pltpu.sync_copy(acc_shared.at[pl.ds(s * NPS, NPS)],
                    out_hbm.at[c, pl.ds(s * NPS, NPS)])


@functools.lru_cache(maxsize=1)
def _edge_agg():
    return pl.kernel(
        _edge_agg_body,
        out_type=jax.ShapeDtypeStruct((NC, N_ACC, D_IN), jnp.float32),
        mesh=plsc.VectorSubcoreMesh(core_axis_name="c", subcore_axis_name="s",
                                    num_cores=NC, num_subcores=NS),
        scratch_types=[
            pltpu.VMEM_SHARED((N_ACC, D_IN), jnp.float32),
            pltpu.VMEM((IDXROWS, CHUNK), jnp.int32),
            pltpu.VMEM((IDXROWS, CHUNK), jnp.int32),
            pltpu.VMEM((CHUNK, D_IN), jnp.float32),
            pltpu.VMEM((CHUNK, D_IN), jnp.float32),
            pltpu.SemaphoreType.DMA,
            pltpu.SemaphoreType.DMA,
        ],
    )


BLK = 1000
GRID = N // BLK


def _bdot(a, b):
    return jnp.dot(a.astype(jnp.bfloat16), b.astype(jnp.bfloat16),
                   preferred_element_type=jnp.float32)


def _mlp_pool_head_body(v_ref, a0_ref, a1_ref, batch_ref, w1_ref, b1_ref,
                        w2_ref, b2_ref, wl1_ref, bl1_ref, gamma_ref, beta_ref,
                        wa_ref, ba_ref, wc_ref, bc_ref, av_ref,
                        out_ref, sums_ref, counts_ref):
    i = pl.program_id(0)
    x = v_ref[...] + a0_ref[...] + a1_ref[...]
    h1 = jnp.maximum(_bdot(x, w1_ref[...]) + b1_ref[...], 0.0)
    h = _bdot(h1, w2_ref[...]) + b2_ref[...]
    b = batch_ref[...].reshape(1, BLK)
    gid = lax.broadcasted_iota(jnp.int32, (G, BLK), 0)
    et = (gid == b).astype(jnp.float32)
    part = jnp.dot(et, h, preferred_element_type=jnp.float32,
                precision=lax.Precision.HIGHEST)
    cpart = jnp.sum(et, axis=1, keepdims=True)

    @pl.when(i == 0)
    def _():
        sums_ref[...] = part
        counts_ref[...] = cpart

    @pl.when(i > 0)
    def _():
        sums_ref[...] += part
        counts_ref[...] += cpart

    @pl.when(i == GRID - 1)
    def _():
        pooled = sums_ref[...] / jnp.maximum(counts_ref[...], 1.0)
        x2 = _bdot(pooled, wl1_ref[...]) + bl1_ref[...]
        mu = jnp.mean(x2, axis=0, keepdims=True)
        var = jnp.mean((x2 - mu) ** 2, axis=0, keepdims=True)
        x2 = gamma_ref[...] * (x2 - mu) / jnp.sqrt(var + 1e-5) + beta_ref[...]
        x2 = jnp.maximum(x2, 0.0)
        x3 = _bdot(x2, wa_ref[...]) + ba_ref[...]
        x3 = av_ref[...] * x3
        out_ref[...] = _bdot(x3, wc_ref[...]) + bc_ref[...]


def _full(shape):
    return pl.BlockSpec(shape, lambda i: tuple(0 for _ in shape))


_mlp_pool_head = pl.pallas_call(
    _mlp_pool_head_body,
    grid=(GRID,),
    in_specs=[
        pl.BlockSpec((BLK, D_IN), lambda i: (i, 0)),
        pl.BlockSpec((BLK, D_IN), lambda i: (i, 0)),
        pl.BlockSpec((BLK, D_IN), lambda i: (i, 0)),
        pl.BlockSpec((1, 1, BLK), lambda i: (i, 0, 0)),
        _full((D_IN, D_HID)),
        _full((1, D_HID)),
        _full((D_HID, D_HID)),
        _full((1, D_HID)),
        _full((D_HID, G)),
        _full((1, G)),
        _full((1, G)),
        _full((1, G)),
        _full((G, 3)),
        _full((1, 3)),
        _full((3, 1)),
        _full((1, 1)),
        _full((G, 3)),
    ],
    out_specs=pl.BlockSpec((G, 1), lambda i: (0, 0)),
    out_shape=jax.ShapeDtypeStruct((G, 1), jnp.float32),
    scratch_shapes=[
        pltpu.VMEM((G, D_HID), jnp.float32),
        pltpu.VMEM((G, 1), jnp.float32),
    ],
)


def kernel(v, attack_vector, edges, batch, W1, b1, W2, b2, Wl1, bl1,
           gamma, beta, Wa, ba, Wc, bc):
    src, dst = edges[0], edges[1]
    def _phase_blocks(flat, fill):
        a = jnp.pad(flat.reshape(NW, EPT), ((0, 0), (0, EPT_PAD - EPT)),
                    constant_values=fill).reshape(NW, NIDX, CHUNK)
        return jnp.stack(
            [a[:, :IDXROWS], a[:, PHCH:PHCH + IDXROWS]], axis=1
        ).reshape(NW * 2, IDXROWS, CHUNK)

    src_p = _phase_blocks(src, 0)
    dst_p = _phase_blocks(dst, N)
    zeros = jnp.zeros((NPS, D_IN), jnp.float32)
    aggp = _edge_agg()(v, src_p, dst_p, zeros)
    return _mlp_pool_head(
        v, aggp[0], aggp[1], batch.reshape(GRID, 1, BLK),
        W1, b1.reshape(1, D_HID), W2, b2.reshape(1, D_HID),
        Wl1, bl1.reshape(1, G), gamma.reshape(1, G), beta.reshape(1, G),
        Wa, ba.reshape(1, 3), Wc, bc.reshape(1, 1), attack_vector)


# speedup vs baseline: 1.0456x; 1.0456x over previous
import functools

import jax
import jax.numpy as jnp
from jax import lax
from jax.experimental import pallas as pl
from jax.experimental.pallas import tpu as pltpu
from jax.experimental.pallas import tpu_sc as plsc

N, E, G, D_IN, D_HID = 10000, 320000, 256, 128, 512

NC, NS = 2, 16
NW = NC * NS
CHUNK = 128
EPT = E // NW
NCHUNK = 80
PHCH = 40
IDXROWS = 48
NIDX = 88
EPT_PAD = NIDX * CHUNK
E_PAD = EPT_PAD * NW
N_ACC = 10112
NPS = N_ACC // NS


def _edge_agg_body(v_hbm, src_hbm, dst_hbm, zeros_hbm, out_hbm,
                   acc_shared, src_idx, dst_idx, rows0, rows1, sem0, sem1):
    c = lax.axis_index("c")
    s = lax.axis_index("s")
    wid = s * NC + c
    pltpu.sync_copy(zeros_hbm, acc_shared.at[pl.ds(s * NPS, NPS)])
    plsc.subcore_barrier()

    for p in range(2):
        pltpu.sync_copy(src_hbm.at[wid * 2 + p], src_idx)
        pltpu.sync_copy(dst_hbm.at[wid * 2 + p], dst_idx)
        pltpu.async_copy(v_hbm.at[src_idx.at[0]], rows0, sem0)

        def step(t, carry):
            j = 2 * t
            pltpu.async_copy(v_hbm.at[src_idx.at[j + 1]], rows1, sem1)
            pltpu.make_async_copy(v_hbm.at[src_idx.at[j]], rows0, sem0).wait()
            pltpu.sync_copy(rows0, acc_shared.at[dst_idx.at[j]], add=True)
            pltpu.async_copy(v_hbm.at[src_idx.at[j + 2]], rows0, sem0)
            pltpu.make_async_copy(v_hbm.at[src_idx.at[j + 1]], rows1,
                                  sem1).wait()
            pltpu.sync_copy(rows1, acc_shared.at[dst_idx.at[j + 1]], add=True)
            return carry

        lax.fori_loop(0, PHCH // 2, step, 0)
        pltpu.make_async_copy(v_hbm.at[src_idx.at[PHCH]], rows0, sem0).wait()

    plsc.subcore_barrier()
    pltpu.sync_copy(acc_shared.at[pl.ds(s * NPS, NPS)],
                    out_hbm.at[c, pl.ds(s * NPS, NPS)])


@functools.lru_cache(maxsize=1)
def _edge_agg():
    return pl.kernel(
        _edge_agg_body,
        out_type=jax.ShapeDtypeStruct((NC, N_ACC, D_IN), jnp.float32),
        mesh=plsc.VectorSubcoreMesh(core_axis_name="c", subcore_axis_name="s",
                                    num_cores=NC, num_subcores=NS),
        scratch_types=[
            pltpu.VMEM_SHARED((N_ACC, D_IN), jnp.float32),
            pltpu.VMEM((IDXROWS, CHUNK), jnp.int32),
            pltpu.VMEM((IDXROWS, CHUNK), jnp.int32),
            pltpu.VMEM((CHUNK, D_IN), jnp.float32),
            pltpu.VMEM((CHUNK, D_IN), jnp.float32),
            pltpu.SemaphoreType.DMA,
            pltpu.SemaphoreType.DMA,
        ],
    )


BLK = 1000
GRID = N // BLK


def _bdot(a, b):
    return jnp.dot(a.astype(jnp.bfloat16), b.astype(jnp.bfloat16),
                   preferred_element_type=jnp.float32)


def _mlp_pool_head_body(v_ref, agg_ref, batch_ref, w1_ref, b1_ref,
                        w2_ref, b2_ref, wl1_ref, bl1_ref, gamma_ref, beta_ref,
                        wa_ref, ba_ref, wc_ref, bc_ref, av_ref,
                        out_ref, sums_ref, counts_ref):
    i = pl.program_id(0)
    x = v_ref[...] + agg_ref[0] + agg_ref[1]
    h1 = jnp.maximum(_bdot(x, w1_ref[...]) + b1_ref[...], 0.0)
    h = _bdot(h1, w2_ref[...]) + b2_ref[...]
    b = batch_ref[...].reshape(1, BLK)
    gid = lax.broadcasted_iota(jnp.int32, (G, BLK), 0)
    et = (gid == b).astype(jnp.float32)
    part = jnp.dot(et, h, preferred_element_type=jnp.float32,
                precision=lax.Precision.HIGHEST)
    cpart = jnp.sum(et, axis=1, keepdims=True)

    @pl.when(i == 0)
    def _():
        sums_ref[...] = part
        counts_ref[...] = cpart

    @pl.when(i > 0)
    def _():
        sums_ref[...] += part
        counts_ref[...] += cpart

    @pl.when(i == GRID - 1)
    def _():
        pooled = sums_ref[...] / jnp.maximum(counts_ref[...], 1.0)
        x2 = _bdot(pooled, wl1_ref[...]) + bl1_ref[...]
        mu = jnp.mean(x2, axis=0, keepdims=True)
        var = jnp.mean((x2 - mu) ** 2, axis=0, keepdims=True)
        x2 = gamma_ref[...] * (x2 - mu) / jnp.sqrt(var + 1e-5) + beta_ref[...]
        x2 = jnp.maximum(x2, 0.0)
        x3 = _bdot(x2, wa_ref[...]) + ba_ref[...]
        x3 = av_ref[...] * x3
        out_ref[...] = _bdot(x3, wc_ref[...]) + bc_ref[...]


def _full(shape):
    return pl.BlockSpec(shape, lambda i: tuple(0 for _ in shape))


_mlp_pool_head = pl.pallas_call(
    _mlp_pool_head_body,
    grid=(GRID,),
    in_specs=[
        pl.BlockSpec((BLK, D_IN), lambda i: (i, 0)),
        pl.BlockSpec((NC, BLK, D_IN), lambda i: (0, i, 0)),
        pl.BlockSpec((1, 1, BLK), lambda i: (i, 0, 0)),
        _full((D_IN, D_HID)),
        _full((1, D_HID)),
        _full((D_HID, D_HID)),
        _full((1, D_HID)),
        _full((D_HID, G)),
        _full((1, G)),
        _full((1, G)),
        _full((1, G)),
        _full((G, 3)),
        _full((1, 3)),
        _full((3, 1)),
        _full((1, 1)),
        _full((G, 3)),
    ],
    out_specs=pl.BlockSpec((G, 1), lambda i: (0, 0)),
    out_shape=jax.ShapeDtypeStruct((G, 1), jnp.float32),
    scratch_shapes=[
        pltpu.VMEM((G, D_HID), jnp.float32),
        pltpu.VMEM((G, 1), jnp.float32),
    ],
)


def kernel(v, attack_vector, edges, batch, W1, b1, W2, b2, Wl1, bl1,
           gamma, beta, Wa, ba, Wc, bc):
    src, dst = edges[0], edges[1]
    def _phase_blocks(flat, fill):
        a = jnp.pad(flat.reshape(NW, EPT), ((0, 0), (0, EPT_PAD - EPT)),
                    constant_values=fill).reshape(NW, NIDX, CHUNK)
        return jnp.stack(
            [a[:, :IDXROWS], a[:, PHCH:PHCH + IDXROWS]], axis=1
        ).reshape(NW * 2, IDXROWS, CHUNK)

    src_p = _phase_blocks(src, 0)
    dst_p = _phase_blocks(dst, N)
    zeros = jnp.zeros((NPS, D_IN), jnp.float32)
    aggp = _edge_agg()(v, src_p, dst_p, zeros)
    return _mlp_pool_head(
        v, aggp, batch.reshape(GRID, 1, BLK),
        W1, b1.reshape(1, D_HID), W2, b2.reshape(1, D_HID),
        Wl1, bl1.reshape(1, G), gamma.reshape(1, G), beta.reshape(1, G),
        Wa, ba.reshape(1, 3), Wc, bc.reshape(1, 1), attack_vector)


# speedup vs baseline: 1.6079x; 1.5378x over previous
import functools

import jax
import jax.numpy as jnp
from jax import lax
from jax.experimental import pallas as pl
from jax.experimental.pallas import tpu as pltpu
from jax.experimental.pallas import tpu_sc as plsc

N, E, G, D_IN, D_HID = 10000, 320000, 256, 128, 512

NC, NS = 2, 16
NW = NC * NS
CHUNK = 128
EPT = E // NW
NCHUNK = -(-EPT // CHUNK)
EPT_PAD = NCHUNK * CHUNK
E_PAD = EPT_PAD * NW
N_ACC = 10112
NPS = N_ACC // NS


def _edge_agg_body(v_hbm, src_hbm, dst_hbm, zeros_hbm, out_hbm,
                   acc_shared, src_idx, dst_idx, rows, sem):
    c = lax.axis_index("c")
    s = lax.axis_index("s")
    wid = s * NC + c
    pltpu.sync_copy(zeros_hbm, acc_shared.at[pl.ds(s * NPS, NPS)])
    pltpu.sync_copy(src_hbm.at[wid], src_idx)
    pltpu.sync_copy(dst_hbm.at[wid], dst_idx)
    plsc.subcore_barrier()

    def step(j, carry):
        pltpu.async_copy(v_hbm.at[src_idx.at[j]], rows, sem).wait()
        pltpu.sync_copy(rows, acc_shared.at[dst_idx.at[j]], add=True)
        return carry

    lax.fori_loop(0, NCHUNK, step, 0)
    plsc.subcore_barrier()
    pltpu.sync_copy(acc_shared.at[pl.ds(s * NPS, NPS)],
                    out_hbm.at[c, pl.ds(s * NPS, NPS)])


@functools.lru_cache(maxsize=1)
def _edge_agg():
    return pl.kernel(
        _edge_agg_body,
        out_type=jax.ShapeDtypeStruct((NC, N_ACC, D_IN), jnp.float32),
        mesh=plsc.VectorSubcoreMesh(core_axis_name="c", subcore_axis_name="s",
                                    num_cores=NC, num_subcores=NS),
        scratch_types=[
            pltpu.VMEM_SHARED((N_ACC, D_IN), jnp.float32),
            pltpu.VMEM((NCHUNK, CHUNK), jnp.int32),
            pltpu.VMEM((NCHUNK, CHUNK), jnp.int32),
            pltpu.VMEM((CHUNK, D_IN), jnp.float32),
            pltpu.SemaphoreType.DMA,
        ],
    )


BLK = 1000
GRID = N // BLK


def _bdot(a, b):
    return jnp.dot(a.astype(jnp.bfloat16), b.astype(jnp.bfloat16),
                   preferred_element_type=jnp.float32)


def _mlp_pool_head_body(v_ref, agg_ref, batch_ref, w1_ref, b1_ref,
                        w2_ref, b2_ref, wl1_ref, bl1_ref, gamma_ref, beta_ref,
                        wa_ref, ba_ref, wc_ref, bc_ref, av_ref,
                        out_ref, sums_ref, counts_ref):
    i = pl.program_id(0)
    x = v_ref[...] + agg_ref[0] + agg_ref[1]
    h1 = jnp.maximum(_bdot(x, w1_ref[...]) + b1_ref[...], 0.0)
    h = _bdot(h1, w2_ref[...]) + b2_ref[...]
    b = batch_ref[...].reshape(1, BLK)
    gid = lax.broadcasted_iota(jnp.int32, (G, BLK), 0)
    et = (gid == b).astype(jnp.float32)
    part = jnp.dot(et, h, preferred_element_type=jnp.float32,
                precision=lax.Precision.HIGHEST)
    cpart = jnp.sum(et, axis=1, keepdims=True)

    @pl.when(i == 0)
    def _():
        sums_ref[...] = part
        counts_ref[...] = cpart

    @pl.when(i > 0)
    def _():
        sums_ref[...] += part
        counts_ref[...] += cpart

    @pl.when(i == GRID - 1)
    def _():
        pooled = sums_ref[...] / jnp.maximum(counts_ref[...], 1.0)
        x2 = _bdot(pooled, wl1_ref[...]) + bl1_ref[...]
        mu = jnp.mean(x2, axis=0, keepdims=True)
        var = jnp.mean((x2 - mu) ** 2, axis=0, keepdims=True)
        x2 = gamma_ref[...] * (x2 - mu) / jnp.sqrt(var + 1e-5) + beta_ref[...]
        x2 = jnp.maximum(x2, 0.0)
        x3 = _bdot(x2, wa_ref[...]) + ba_ref[...]
        x3 = av_ref[...] * x3
        out_ref[...] = _bdot(x3, wc_ref[...]) + bc_ref[...]


def _full(shape):
    return pl.BlockSpec(shape, lambda i: tuple(0 for _ in shape))


_mlp_pool_head = pl.pallas_call(
    _mlp_pool_head_body,
    grid=(GRID,),
    in_specs=[
        pl.BlockSpec((BLK, D_IN), lambda i: (i, 0)),
        pl.BlockSpec((NC, BLK, D_IN), lambda i: (0, i, 0)),
        pl.BlockSpec((1, 1, BLK), lambda i: (i, 0, 0)),
        _full((D_IN, D_HID)),
        _full((1, D_HID)),
        _full((D_HID, D_HID)),
        _full((1, D_HID)),
        _full((D_HID, G)),
        _full((1, G)),
        _full((1, G)),
        _full((1, G)),
        _full((G, 3)),
        _full((1, 3)),
        _full((3, 1)),
        _full((1, 1)),
        _full((G, 3)),
    ],
    out_specs=pl.BlockSpec((G, 1), lambda i: (0, 0)),
    out_shape=jax.ShapeDtypeStruct((G, 1), jnp.float32),
    scratch_shapes=[
        pltpu.VMEM((G, D_HID), jnp.float32),
        pltpu.VMEM((G, 1), jnp.float32),
    ],
)


def kernel(v, attack_vector, edges, batch, W1, b1, W2, b2, Wl1, bl1,
           gamma, beta, Wa, ba, Wc, bc):
    src, dst = edges[0], edges[1]
    pad = E_PAD - E
    src_p = jnp.concatenate(
        [src, jnp.zeros((pad,), jnp.int32)]).reshape(NW, NCHUNK, CHUNK)
    dst_p = jnp.concatenate(
        [dst, jnp.full((pad,), N, jnp.int32)]).reshape(NW, NCHUNK, CHUNK)
    zeros = jnp.zeros((NPS, D_IN), jnp.float32)
    aggp = _edge_agg()(v, src_p, dst_p, zeros)
    return _mlp_pool_head(
        v, aggp, batch.reshape(GRID, 1, BLK),
        W1, b1.reshape(1, D_HID), W2, b2.reshape(1, D_HID),
        Wl1, bl1.reshape(1, G), gamma.reshape(1, G), beta.reshape(1, G),
        Wa, ba.reshape(1, 3), Wc, bc.reshape(1, 1), attack_vector)
